# baseline (speedup 1.0000x reference)
import jax
import jax.numpy as jnp
from jax.experimental import pallas as pl
from jax.experimental.pallas import tpu as pltpu

_NUM_RANKS = 1024
_MAX_TOK = 77
_D = 768
_CTX = 16
_TPR = 4
_CB = 64
_TAIL = 1 + _CTX + _TPR


def _body(ctx_ref, rank_hbm, sent_hbm, out_hbm, ctx_big, sem_a, sem_b,
          sem_c, sem_d):
    a = pltpu.make_async_copy(
        sent_hbm.at[:, pl.ds(_TAIL * _D, (_MAX_TOK - _TAIL) * _D)],
        out_hbm.at[:, pl.ds(_TAIL * _D, (_MAX_TOK - _TAIL) * _D)], sem_a)
    a.start()
    b = pltpu.make_async_copy(
        sent_hbm.at[:, pl.ds(0, _D)], out_hbm.at[:, pl.ds(0, _D)], sem_b)
    b.start()
    c = pltpu.make_async_copy(
        rank_hbm, out_hbm.at[:, pl.ds((1 + _CTX) * _D, _TPR * _D)], sem_c)
    c.start()
    ctx_big[...] = jnp.broadcast_to(ctx_ref[...], (_CB, _CTX * _D))
    for i in range(_NUM_RANKS // _CB):
        pltpu.make_async_copy(
            ctx_big, out_hbm.at[pl.ds(i * _CB, _CB), pl.ds(_D, _CTX * _D)],
            sem_d).start()
    for i in range(_NUM_RANKS // _CB):
        pltpu.make_async_copy(
            ctx_big, out_hbm.at[pl.ds(i * _CB, _CB), pl.ds(_D, _CTX * _D)],
            sem_d).wait()
    a.wait()
    b.wait()
    c.wait()


def kernel(context_embeds, rank_embeds, sentence_embeds):
    ctx2 = context_embeds.reshape(1, _CTX * _D)
    rank2 = rank_embeds.reshape(_NUM_RANKS, _TPR * _D)
    sent2 = sentence_embeds.reshape(_NUM_RANKS, _MAX_TOK * _D)
    out2 = pl.pallas_call(
        _body,
        in_specs=[
            pl.BlockSpec(memory_space=pltpu.VMEM),
            pl.BlockSpec(memory_space=pltpu.MemorySpace.HBM),
            pl.BlockSpec(memory_space=pltpu.MemorySpace.HBM),
        ],
        out_specs=pl.BlockSpec(memory_space=pltpu.MemorySpace.HBM),
        out_shape=jax.ShapeDtypeStruct((_NUM_RANKS, _MAX_TOK * _D),
                                       jnp.float32),
        scratch_shapes=[
            pltpu.VMEM((_CB, _CTX * _D), jnp.float32),
            pltpu.SemaphoreType.DMA,
            pltpu.SemaphoreType.DMA,
            pltpu.SemaphoreType.DMA,
            pltpu.SemaphoreType.DMA,
        ],
    )(ctx2, rank2, sent2)
    return out2.reshape(_NUM_RANKS, _MAX_TOK, _D)


# speedup vs baseline: 12.6322x; 12.6322x over previous
import jax
import jax.numpy as jnp
from jax.experimental import pallas as pl

_NUM_RANKS = 1024
_MAX_TOK = 77
_D = 768
_CTX = 16
_TPR = 4
_RB = 32


def _body(ctx_ref, rank_ref, sent_ref, out_ref):
    out_ref[:, 0:1, :] = sent_ref[:, 0:1, :]
    out_ref[:, 1:1 + _CTX, :] = jnp.broadcast_to(ctx_ref[...][None],
                                                 (_RB, _CTX, _D))
    out_ref[:, 1 + _CTX:1 + _CTX + _TPR, :] = rank_ref[...]
    tail = 1 + _CTX + _TPR
    out_ref[:, tail:, :] = sent_ref[:, tail:, :]


def kernel(context_embeds, rank_embeds, sentence_embeds):
    return pl.pallas_call(
        _body,
        grid=(_NUM_RANKS // _RB,),
        in_specs=[
            pl.BlockSpec((_CTX, _D), lambda i: (0, 0)),
            pl.BlockSpec((_RB, _TPR, _D), lambda i: (i, 0, 0)),
            pl.BlockSpec((_RB, _MAX_TOK, _D), lambda i: (i, 0, 0)),
        ],
        out_specs=pl.BlockSpec((_RB, _MAX_TOK, _D), lambda i: (i, 0, 0)),
        out_shape=jax.ShapeDtypeStruct((_NUM_RANKS, _MAX_TOK, _D),
                                       jnp.float32),
    )(context_embeds, rank_embeds, sentence_embeds)
